# baseline (speedup 1.0000x reference)
import dataclasses
import functools

import jax
import jax.numpy as jnp
from jax import lax
from jax.experimental import pallas as pl
from jax.experimental.pallas import tpu as pltpu
from jax.experimental.pallas import tpu_sc as plsc

E = 160000
NSUB = 20000
B = 128
D = 256
DS = 128

NPAD = 20480
NC, NS, L = 2, 16, 16
NW = NC * NS
PER_TILE = 5120
EP = NW * PER_TILE
GCHUNK = 128
NCH = PER_TILE // L

EH = E // 2
PER_TILE_H = 2560
EP_H = NW * PER_TILE_H
NGCH_H = PER_TILE_H // GCHUNK

_mesh = plsc.VectorSubcoreMesh(core_axis_name="c", subcore_axis_name="s")

_sc_params = pltpu.CompilerParams()
if "needs_layout_passes" in pltpu.CompilerParams.__dataclass_fields__:
    _sc_params = dataclasses.replace(_sc_params, needs_layout_passes=False)


def _leaky(x):
    return jnp.where(x >= 0, x, 0.01 * x)


def _node_tables_body(mem_ref, w_ref, b_ref, nl_ref, nr_ref, hl_ref):
    x = jnp.dot(mem_ref[...], w_ref[...], preferred_element_type=jnp.float32,
                precision=lax.Precision.HIGHEST)
    x = x + b_ref[...]
    nl_ref[...] = x[:, :DS]
    nr_ref[...] = x[:, DS:2 * DS]
    hl_ref[...] = x[:, 2 * DS:]


def _node_tables(mem_pad, w_node, b_node):
    blk = 2048
    return pl.pallas_call(
        _node_tables_body,
        grid=(NPAD // blk,),
        in_specs=[
            pl.BlockSpec((blk, D), lambda i: (i, 0)),
            pl.BlockSpec((D, 2 * DS + D), lambda i: (0, 0)),
            pl.BlockSpec((1, 2 * DS + D), lambda i: (0, 0)),
        ],
        out_specs=[
            pl.BlockSpec((blk, DS), lambda i: (i, 0)),
            pl.BlockSpec((blk, DS), lambda i: (i, 0)),
            pl.BlockSpec((blk, D), lambda i: (i, 0)),
        ],
        out_shape=[
            jax.ShapeDtypeStruct((NPAD, DS), jnp.float32),
            jax.ShapeDtypeStruct((NPAD, DS), jnp.float32),
            jax.ShapeDtypeStruct((NPAD, D), jnp.float32),
        ],
    )(mem_pad, w_node, b_node)


def _query_tables_body(q_ref, w_ref, c_ref, o_ref):
    o_ref[...] = (
        jnp.dot(q_ref[...], w_ref[...], preferred_element_type=jnp.float32,
                precision=lax.Precision.HIGHEST)
        + c_ref[...]
    )


def _query_tables(qcat, w_q, c_q):
    return pl.pallas_call(
        _query_tables_body,
        out_shape=jax.ShapeDtypeStruct((B, 2 * DS), jnp.float32),
    )(qcat, w_q, c_q)


def _logits_body(rel_ref, g1_ref, g2_ref, eg_ref, wrel_ref, q_ref, wc_ref,
                 bc_ref, out_ref):
    hi = lax.Precision.HIGHEST
    rel = rel_ref[...]
    rl = jnp.dot(rel, wrel_ref[...], preferred_element_type=jnp.float32,
                 precision=hi)
    eg = eg_ref[0, 0, :]
    lane = jax.lax.broadcasted_iota(jnp.int32, (1, B), 1)
    oh = (eg[:, None] == lane).astype(jnp.float32)
    q = jnp.dot(oh, q_ref[...], preferred_element_type=jnp.float32,
                precision=hi)
    lp = _leaky(g1_ref[...] + rl[:, :DS] + q[:, :DS])
    rp = _leaky(g2_ref[...] + rl[:, DS:] + q[:, DS:])
    c = jnp.dot(rp, wc_ref[...], preferred_element_type=jnp.float32,
                precision=hi) + bc_ref[...]
    out_ref[0, 0, :] = jnp.sum(lp * c, axis=1)


def _logits_half(h, rel_emb, g1, g2, eg3d, w_rel, qlqr, wc_t, bc):
    blk = 3200
    nblk = EH // blk
    off = h * nblk
    out = pl.pallas_call(
        _logits_body,
        grid=(nblk,),
        in_specs=[
            pl.BlockSpec((blk, D), lambda i: (i + off, 0)),
            pl.BlockSpec((blk, DS), lambda i: (i, 0)),
            pl.BlockSpec((blk, DS), lambda i: (i, 0)),
            pl.BlockSpec((1, 1, blk), lambda i: (i + off, 0, 0)),
            pl.BlockSpec((D, D), lambda i: (0, 0)),
            pl.BlockSpec((B, 2 * DS), lambda i: (0, 0)),
            pl.BlockSpec((DS, DS), lambda i: (0, 0)),
            pl.BlockSpec((1, DS), lambda i: (0, 0)),
        ],
        out_specs=pl.BlockSpec((1, 1, blk), lambda i: (i, 0, 0)),
        out_shape=jax.ShapeDtypeStruct((nblk, 1, blk), jnp.float32),
    )(rel_emb, g1, g2, eg3d, w_rel, qlqr, wc_t, bc)
    return out.reshape(EH)


def _combine_body(op, p_ref, o_ref):
    o_ref[...] = op(p_ref[...], axis=0)


def _combine(partials, op):
    blk = 4096
    return pl.pallas_call(
        functools.partial(_combine_body, op),
        grid=(NPAD // blk,),
        in_specs=[pl.BlockSpec((NW, blk), lambda i: (0, i))],
        out_specs=pl.BlockSpec((blk,), lambda i: (i,)),
        out_shape=jax.ShapeDtypeStruct((NPAD,), jnp.float32),
    )(partials)


def _final_body(ps_ref, pw_ref, hl_ref, o1_ref, o2_ref):
    s = jnp.sum(ps_ref[...], axis=0)
    o1_ref[...] = jnp.sum(pw_ref[...], axis=0)
    o2_ref[...] = s[:, None] * hl_ref[...]


def _final(partials_s, partials_w, hl):
    blk = 4096
    return pl.pallas_call(
        _final_body,
        grid=(NPAD // blk,),
        in_specs=[
            pl.BlockSpec((NW, blk), lambda i: (0, i)),
            pl.BlockSpec((NW, blk), lambda i: (0, i)),
            pl.BlockSpec((blk, D), lambda i: (i, 0)),
        ],
        out_specs=[
            pl.BlockSpec((blk,), lambda i: (i,)),
            pl.BlockSpec((blk, D), lambda i: (i, 0)),
        ],
        out_shape=[
            jax.ShapeDtypeStruct((NPAD,), jnp.float32),
            jax.ShapeDtypeStruct((NPAD, D), jnp.float32),
        ],
    )(partials_s, partials_w, hl)


def _wid():
    return lax.axis_index("s") * NC + lax.axis_index("c")


def _fill(ref, n, value):
    v = jnp.full((L,), value, dtype=ref.dtype)

    @pl.loop(0, n // L)
    def _(i):
        ref[pl.ds(i * L, L)] = v


def _rmw_scatter(acc, idx, vals, op):
    lane = lax.iota(jnp.int32, L)
    for k in range(L):
        g = plsc.load_gather(acc, [idx])
        plsc.store_scatter(acc, [idx], op(g, vals), mask=lane == k)


def _make_sc_gather():
    @functools.partial(
        pl.kernel,
        mesh=_mesh,
        out_type=[
            jax.ShapeDtypeStruct((EP_H, DS), jnp.float32),
            jax.ShapeDtypeStruct((EP_H, DS), jnp.float32),
        ],
        scratch_types=[
            pltpu.VMEM((PER_TILE_H,), jnp.int32),
            pltpu.VMEM((PER_TILE_H,), jnp.int32),
            pltpu.VMEM((GCHUNK, DS), jnp.float32),
            pltpu.VMEM((GCHUNK, DS), jnp.float32),
            pltpu.VMEM((GCHUNK, DS), jnp.float32),
            pltpu.VMEM((GCHUNK, DS), jnp.float32),
            pltpu.SemaphoreType.DMA,
            pltpu.SemaphoreType.DMA,
            pltpu.SemaphoreType.DMA,
            pltpu.SemaphoreType.DMA,
        ],
    )
    def _gather(nl_hbm, nr_hbm, ivi_hbm, ivj_hbm, g1_hbm, g2_hbm,
                ivi_buf, ivj_buf, r1a, r2a, r1b, r2b, sga, sgb, swa, swb):
        base = _wid() * PER_TILE_H
        pltpu.sync_copy(ivi_hbm.at[pl.ds(base, PER_TILE_H)], ivi_buf)
        pltpu.sync_copy(ivj_hbm.at[pl.ds(base, PER_TILE_H)], ivj_buf)

        def start_g(c, r1, r2, sg):
            pltpu.async_copy(nl_hbm.at[ivi_buf.at[pl.ds(c * GCHUNK, GCHUNK)]],
                             r1, sg)
            pltpu.async_copy(nr_hbm.at[ivj_buf.at[pl.ds(c * GCHUNK, GCHUNK)]],
                             r2, sg)

        def wait2(r1, r2, sem):
            pltpu.make_async_copy(nl_hbm.at[pl.ds(0, GCHUNK)], r1, sem).wait()
            pltpu.make_async_copy(nr_hbm.at[pl.ds(0, GCHUNK)], r2, sem).wait()

        def start_w(c, r1, r2, sw):
            off = base + c * GCHUNK
            pltpu.async_copy(r1, g1_hbm.at[pl.ds(off, GCHUNK)], sw)
            pltpu.async_copy(r2, g2_hbm.at[pl.ds(off, GCHUNK)], sw)

        start_g(0, r1a, r2a, sga)

        @pl.loop(0, NGCH_H // 2)
        def _(i):
            a = 2 * i

            @pl.when(i > 0)
            def _():
                wait2(r1b, r2b, swb)
            start_g(a + 1, r1b, r2b, sgb)
            wait2(r1a, r2a, sga)
            start_w(a, r1a, r2a, swa)
            wait2(r1a, r2a, swa)

            @pl.when(i < NGCH_H // 2 - 1)
            def _():
                start_g(a + 2, r1a, r2a, sga)
            wait2(r1b, r2b, sgb)
            start_w(a + 1, r1b, r2b, swb)

        wait2(r1b, r2b, swb)

    return _gather


_sc_gather_half = _make_sc_gather()


@functools.partial(
    pl.kernel,
    mesh=_mesh,
    compiler_params=_sc_params,
    out_type=jax.ShapeDtypeStruct((NW, NPAD), jnp.float32),
    scratch_types=[
        pltpu.VMEM((PER_TILE,), jnp.float32),
        pltpu.VMEM((PER_TILE,), jnp.int32),
        pltpu.VMEM((NPAD,), jnp.float32),
    ],
)
def _sc_segmax(logits_hbm, ivi_hbm, out_hbm, lbuf, ibuf, acc):
    wid = _wid()
    base = wid * PER_TILE
    _fill(acc, NPAD, -1e30)
    pltpu.sync_copy(logits_hbm.at[pl.ds(base, PER_TILE)], lbuf)
    pltpu.sync_copy(ivi_hbm.at[pl.ds(base, PER_TILE)], ibuf)

    @pl.loop(0, NCH)
    def _(c):
        iv = ibuf[pl.ds(c * L, L)]
        lv = lbuf[pl.ds(c * L, L)]
        _rmw_scatter(acc, iv, lv, jnp.maximum)

    pltpu.sync_copy(acc, out_hbm.at[wid])


@functools.partial(
    pl.kernel,
    mesh=_mesh,
    compiler_params=_sc_params,
    out_type=[
        jax.ShapeDtypeStruct((EP,), jnp.float32),
        jax.ShapeDtypeStruct((NW, NPAD), jnp.float32),
    ],
    scratch_types=[
        pltpu.VMEM((NPAD,), jnp.float32),
        pltpu.VMEM((PER_TILE,), jnp.float32),
        pltpu.VMEM((PER_TILE,), jnp.int32),
        pltpu.VMEM((PER_TILE,), jnp.float32),
        pltpu.VMEM((NPAD,), jnp.float32),
    ],
)
def _sc_denom(logits_hbm, ivi_hbm, segmax_hbm, ex_hbm, out_hbm,
              smtab, lbuf, ibuf, exbuf, acc):
    wid = _wid()
    base = wid * PER_TILE
    pltpu.sync_copy(segmax_hbm, smtab)
    _fill(acc, NPAD, 0.0)
    pltpu.sync_copy(logits_hbm.at[pl.ds(base, PER_TILE)], lbuf)
    pltpu.sync_copy(ivi_hbm.at[pl.ds(base, PER_TILE)], ibuf)

    @pl.loop(0, NCH)
    def _(c):
        iv = ibuf[pl.ds(c * L, L)]
        lv = lbuf[pl.ds(c * L, L)]
        m = plsc.load_gather(smtab, [iv])
        e = jnp.exp(lv - m)
        exbuf[pl.ds(c * L, L)] = e
        _rmw_scatter(acc, iv, e, jnp.add)

    pltpu.sync_copy(exbuf, ex_hbm.at[pl.ds(base, PER_TILE)])
    pltpu.sync_copy(acc, out_hbm.at[wid])


@functools.partial(
    pl.kernel,
    mesh=_mesh,
    compiler_params=_sc_params,
    out_type=[
        jax.ShapeDtypeStruct((NW, NPAD), jnp.float32),
        jax.ShapeDtypeStruct((NW, NPAD), jnp.float32),
    ],
    scratch_types=[
        pltpu.VMEM((NPAD,), jnp.float32),
        pltpu.VMEM((PER_TILE,), jnp.float32),
        pltpu.VMEM((PER_TILE,), jnp.float32),
        pltpu.VMEM((PER_TILE,), jnp.int32),
        pltpu.VMEM((PER_TILE,), jnp.int32),
        pltpu.VMEM((NPAD,), jnp.float32),
        pltpu.VMEM((NPAD,), jnp.float32),
    ],
)
def _sc_scatter_vj(ex_hbm, na_hbm, ivi_hbm, ivj_hbm, denom_hbm,
                   outs_hbm, outw_hbm,
                   dtab, exb, nab, ibi, ibj, acc_s, acc_w):
    wid = _wid()
    base = wid * PER_TILE
    pltpu.sync_copy(denom_hbm, dtab)
    _fill(acc_s, NPAD, 0.0)
    _fill(acc_w, NPAD, 0.0)
    pltpu.sync_copy(ex_hbm.at[pl.ds(base, PER_TILE)], exb)
    pltpu.sync_copy(na_hbm.at[pl.ds(base, PER_TILE)], nab)
    pltpu.sync_copy(ivi_hbm.at[pl.ds(base, PER_TILE)], ibi)
    pltpu.sync_copy(ivj_hbm.at[pl.ds(base, PER_TILE)], ibj)

    @pl.loop(0, NCH)
    def _(c):
        ivi = ibi[pl.ds(c * L, L)]
        ivj = ibj[pl.ds(c * L, L)]
        ev = exb[pl.ds(c * L, L)]
        nav = nab[pl.ds(c * L, L)]
        d = plsc.load_gather(dtab, [ivi])
        t = ev / (d + 1e-20)
        w = t * nav
        _rmw_scatter(acc_s, ivj, t, jnp.add)
        _rmw_scatter(acc_w, ivj, w, jnp.add)

    pltpu.sync_copy(acc_s, outs_hbm.at[wid])
    pltpu.sync_copy(acc_w, outw_hbm.at[wid])


def kernel(node_attention, memorized_embedding, rel_emb, query_src_emb,
           query_rel_emb, query_time_emb, edge_eg, idx_vi, idx_vj,
           W_proj, b_proj, W_left, b_left, W_right, b_right,
           W_center, b_center, W_lin, b_lin):
    f32 = jnp.float32
    P = W_proj
    Lb = [W_left[:, i * DS:(i + 1) * DS] for i in range(5)]
    Rb = [W_right[:, i * DS:(i + 1) * DS] for i in range(5)]

    w_node = jnp.concatenate([P.T @ Lb[0].T, P.T @ Rb[0].T, W_lin.T], axis=1)
    b_node = jnp.concatenate(
        [jnp.zeros((2 * DS,), f32), b_lin]).reshape(1, 2 * DS + D)
    w_rel = jnp.concatenate([P.T @ Lb[1].T, P.T @ Rb[1].T], axis=1)
    w_q = jnp.concatenate([
        jnp.concatenate([P.T @ Lb[2].T, P.T @ Rb[2].T], axis=1),
        jnp.concatenate([P.T @ Lb[3].T, P.T @ Rb[3].T], axis=1),
        jnp.concatenate([P.T @ Lb[4].T, P.T @ Rb[4].T], axis=1),
    ], axis=0)
    c_l = b_left + b_proj @ (Lb[0].T + Lb[1].T + Lb[2].T + Lb[3].T + Lb[4].T)
    c_r = b_right + b_proj @ (Rb[0].T + Rb[1].T + Rb[2].T + Rb[3].T + Rb[4].T)
    c_q = jnp.concatenate([c_l, c_r]).reshape(1, 2 * DS)
    qcat = jnp.concatenate([query_src_emb, query_rel_emb, query_time_emb],
                           axis=1)

    mem_pad = jnp.pad(memorized_embedding, ((0, NPAD - NSUB), (0, 0)))
    pad_e = EP - E
    ivi_p = jnp.concatenate(
        [idx_vi.astype(jnp.int32), jnp.full((pad_e,), NSUB, jnp.int32)])
    ivj_p = jnp.concatenate(
        [idx_vj.astype(jnp.int32), jnp.full((pad_e,), NSUB, jnp.int32)])
    na_p = jnp.concatenate([node_attention, jnp.zeros((pad_e,), f32)])
    eg3d = edge_eg.astype(jnp.int32).reshape(E // 3200, 1, 3200)

    nl_tab, nr_tab, hl_tab = _node_tables(mem_pad, w_node, b_node)
    qlqr = _query_tables(qcat, w_q, c_q)

    pad_h = EP_H - EH
    ivi32 = idx_vi.astype(jnp.int32)
    ivj32 = idx_vj.astype(jnp.int32)
    gs = []
    for h in (0, 1):
        ivi_h = jnp.concatenate([ivi32[h * EH:(h + 1) * EH],
                                 jnp.full((pad_h,), NSUB, jnp.int32)])
        ivj_h = jnp.concatenate([ivj32[h * EH:(h + 1) * EH],
                                 jnp.full((pad_h,), NSUB, jnp.int32)])
        gs.append(_sc_gather_half(nl_tab, nr_tab, ivi_h, ivj_h))
    halves = [
        _logits_half(h, rel_emb, gs[h][0], gs[h][1], eg3d, w_rel, qlqr,
                     W_center.T, b_center.reshape(1, DS))
        for h in (0, 1)
    ]
    logits_p = jnp.concatenate([halves[0], halves[1],
                                jnp.zeros((pad_e,), f32)])

    segmax_parts = _sc_segmax(logits_p, ivi_p)
    segmax = _combine(segmax_parts, jnp.max)
    ex, denom_parts = _sc_denom(logits_p, ivi_p, segmax)
    denom = _combine(denom_parts, jnp.sum)
    parts_s, parts_w = _sc_scatter_vj(ex, na_p, ivi_p, ivj_p, denom)

    out1_p, out2_p = _final(parts_s, parts_w, hl_tab)
    return out1_p[:NSUB], out2_p[:NSUB]


# speedup vs baseline: 1.4100x; 1.4100x over previous
import dataclasses
import functools

import jax
import jax.numpy as jnp
from jax import lax
from jax.experimental import pallas as pl
from jax.experimental.pallas import tpu as pltpu
from jax.experimental.pallas import tpu_sc as plsc

E = 160000
NSUB = 20000
B = 128
D = 256
DS = 128

NPAD = 20480
NC, NS, L = 2, 16, 16
NW = NC * NS
PER_TILE = 5120
EP = NW * PER_TILE
GCHUNK = 128
NCH = PER_TILE // L

EH = E // 2
PER_TILE_H = 2560
EP_H = NW * PER_TILE_H
NGCH_H = PER_TILE_H // GCHUNK

_mesh = plsc.VectorSubcoreMesh(core_axis_name="c", subcore_axis_name="s")

_sc_params = pltpu.CompilerParams()
if "needs_layout_passes" in pltpu.CompilerParams.__dataclass_fields__:
    _sc_params = dataclasses.replace(_sc_params, needs_layout_passes=False)


def _leaky(x):
    return jnp.where(x >= 0, x, 0.01 * x)


def _node_tables_body(mem_ref, w_ref, b_ref, nl_ref, nr_ref, hl_ref):
    x = jnp.dot(mem_ref[...], w_ref[...], preferred_element_type=jnp.float32)
    x = x + b_ref[...]
    nl_ref[...] = x[:, :DS]
    nr_ref[...] = x[:, DS:2 * DS]
    hl_ref[...] = x[:, 2 * DS:]


def _node_tables(mem_pad, w_node, b_node):
    blk = 2048
    return pl.pallas_call(
        _node_tables_body,
        grid=(NPAD // blk,),
        in_specs=[
            pl.BlockSpec((blk, D), lambda i: (i, 0)),
            pl.BlockSpec((D, 2 * DS + D), lambda i: (0, 0)),
            pl.BlockSpec((1, 2 * DS + D), lambda i: (0, 0)),
        ],
        out_specs=[
            pl.BlockSpec((blk, DS), lambda i: (i, 0)),
            pl.BlockSpec((blk, DS), lambda i: (i, 0)),
            pl.BlockSpec((blk, D), lambda i: (i, 0)),
        ],
        out_shape=[
            jax.ShapeDtypeStruct((NPAD, DS), jnp.float32),
            jax.ShapeDtypeStruct((NPAD, DS), jnp.float32),
            jax.ShapeDtypeStruct((NPAD, D), jnp.float32),
        ],
    )(mem_pad, w_node, b_node)


def _query_tables_body(q_ref, w_ref, c_ref, o_ref):
    o_ref[...] = (
        jnp.dot(q_ref[...], w_ref[...], preferred_element_type=jnp.float32)
        + c_ref[...]
    )


def _query_tables(qcat, w_q, c_q):
    return pl.pallas_call(
        _query_tables_body,
        out_shape=jax.ShapeDtypeStruct((B, 2 * DS), jnp.float32),
    )(qcat, w_q, c_q)


def _logits_body(rel_ref, g1_ref, g2_ref, eg_ref, wrel_ref, q_ref, wc_ref,
                 bc_ref, out_ref):
    rel = rel_ref[...]
    rl = jnp.dot(rel, wrel_ref[...], preferred_element_type=jnp.float32)
    eg = eg_ref[0, 0, :]
    lane = jax.lax.broadcasted_iota(jnp.int32, (1, B), 1)
    oh = (eg[:, None] == lane).astype(jnp.float32)
    q = jnp.dot(oh, q_ref[...], preferred_element_type=jnp.float32)
    lp = _leaky(g1_ref[...] + rl[:, :DS] + q[:, :DS])
    rp = _leaky(g2_ref[...] + rl[:, DS:] + q[:, DS:])
    c = jnp.dot(rp, wc_ref[...], preferred_element_type=jnp.float32) + bc_ref[...]
    out_ref[0, 0, :] = jnp.sum(lp * c, axis=1)


def _logits_half(h, rel_emb, g1, g2, eg3d, w_rel, qlqr, wc_t, bc):
    blk = 3200
    nblk = EH // blk
    off = h * nblk
    out = pl.pallas_call(
        _logits_body,
        grid=(nblk,),
        in_specs=[
            pl.BlockSpec((blk, D), lambda i: (i + off, 0)),
            pl.BlockSpec((blk, DS), lambda i: (i, 0)),
            pl.BlockSpec((blk, DS), lambda i: (i, 0)),
            pl.BlockSpec((1, 1, blk), lambda i: (i + off, 0, 0)),
            pl.BlockSpec((D, D), lambda i: (0, 0)),
            pl.BlockSpec((B, 2 * DS), lambda i: (0, 0)),
            pl.BlockSpec((DS, DS), lambda i: (0, 0)),
            pl.BlockSpec((1, DS), lambda i: (0, 0)),
        ],
        out_specs=pl.BlockSpec((1, 1, blk), lambda i: (i, 0, 0)),
        out_shape=jax.ShapeDtypeStruct((nblk, 1, blk), jnp.float32),
    )(rel_emb, g1, g2, eg3d, w_rel, qlqr, wc_t, bc)
    return out.reshape(EH)


def _combine_body(op, p_ref, o_ref):
    o_ref[...] = op(p_ref[...], axis=0)


def _combine(partials, op):
    blk = 4096
    return pl.pallas_call(
        functools.partial(_combine_body, op),
        grid=(NPAD // blk,),
        in_specs=[pl.BlockSpec((NW, blk), lambda i: (0, i))],
        out_specs=pl.BlockSpec((blk,), lambda i: (i,)),
        out_shape=jax.ShapeDtypeStruct((NPAD,), jnp.float32),
    )(partials)


def _final_body(ps_ref, pw_ref, hl_ref, o1_ref, o2_ref):
    s = jnp.sum(ps_ref[...], axis=0)
    o1_ref[...] = jnp.sum(pw_ref[...], axis=0)
    o2_ref[...] = s[:, None] * hl_ref[...]


def _final(partials_s, partials_w, hl):
    blk = 4096
    return pl.pallas_call(
        _final_body,
        grid=(NPAD // blk,),
        in_specs=[
            pl.BlockSpec((NW, blk), lambda i: (0, i)),
            pl.BlockSpec((NW, blk), lambda i: (0, i)),
            pl.BlockSpec((blk, D), lambda i: (i, 0)),
        ],
        out_specs=[
            pl.BlockSpec((blk,), lambda i: (i,)),
            pl.BlockSpec((blk, D), lambda i: (i, 0)),
        ],
        out_shape=[
            jax.ShapeDtypeStruct((NPAD,), jnp.float32),
            jax.ShapeDtypeStruct((NPAD, D), jnp.float32),
        ],
    )(partials_s, partials_w, hl)


def _wid():
    return lax.axis_index("s") * NC + lax.axis_index("c")


def _fill(ref, n, value):
    v = jnp.full((L,), value, dtype=ref.dtype)

    @pl.loop(0, n // L)
    def _(i):
        ref[pl.ds(i * L, L)] = v


def _rmw_scatter(acc, idx, vals, op):
    lane = lax.iota(jnp.int32, L)
    for k in range(L):
        g = plsc.load_gather(acc, [idx])
        plsc.store_scatter(acc, [idx], op(g, vals), mask=lane == k)


def _make_sc_gather():
    @functools.partial(
        pl.kernel,
        mesh=_mesh,
        out_type=[
            jax.ShapeDtypeStruct((EP_H, DS), jnp.float32),
            jax.ShapeDtypeStruct((EP_H, DS), jnp.float32),
        ],
        scratch_types=[
            pltpu.VMEM((PER_TILE_H,), jnp.int32),
            pltpu.VMEM((PER_TILE_H,), jnp.int32),
            pltpu.VMEM((GCHUNK, DS), jnp.float32),
            pltpu.VMEM((GCHUNK, DS), jnp.float32),
            pltpu.VMEM((GCHUNK, DS), jnp.float32),
            pltpu.VMEM((GCHUNK, DS), jnp.float32),
            pltpu.SemaphoreType.DMA,
            pltpu.SemaphoreType.DMA,
            pltpu.SemaphoreType.DMA,
            pltpu.SemaphoreType.DMA,
        ],
    )
    def _gather(nl_hbm, nr_hbm, ivi_hbm, ivj_hbm, g1_hbm, g2_hbm,
                ivi_buf, ivj_buf, r1a, r2a, r1b, r2b, sga, sgb, swa, swb):
        base = _wid() * PER_TILE_H
        pltpu.sync_copy(ivi_hbm.at[pl.ds(base, PER_TILE_H)], ivi_buf)
        pltpu.sync_copy(ivj_hbm.at[pl.ds(base, PER_TILE_H)], ivj_buf)

        def start_g(c, r1, r2, sg):
            pltpu.async_copy(nl_hbm.at[ivi_buf.at[pl.ds(c * GCHUNK, GCHUNK)]],
                             r1, sg)
            pltpu.async_copy(nr_hbm.at[ivj_buf.at[pl.ds(c * GCHUNK, GCHUNK)]],
                             r2, sg)

        def wait2(r1, r2, sem):
            pltpu.make_async_copy(nl_hbm.at[pl.ds(0, GCHUNK)], r1, sem).wait()
            pltpu.make_async_copy(nr_hbm.at[pl.ds(0, GCHUNK)], r2, sem).wait()

        def start_w(c, r1, r2, sw):
            off = base + c * GCHUNK
            pltpu.async_copy(r1, g1_hbm.at[pl.ds(off, GCHUNK)], sw)
            pltpu.async_copy(r2, g2_hbm.at[pl.ds(off, GCHUNK)], sw)

        start_g(0, r1a, r2a, sga)

        @pl.loop(0, NGCH_H // 2)
        def _(i):
            a = 2 * i

            @pl.when(i > 0)
            def _():
                wait2(r1b, r2b, swb)
            start_g(a + 1, r1b, r2b, sgb)
            wait2(r1a, r2a, sga)
            start_w(a, r1a, r2a, swa)
            wait2(r1a, r2a, swa)

            @pl.when(i < NGCH_H // 2 - 1)
            def _():
                start_g(a + 2, r1a, r2a, sga)
            wait2(r1b, r2b, sgb)
            start_w(a + 1, r1b, r2b, swb)

        wait2(r1b, r2b, swb)

    return _gather


_sc_gather_half = _make_sc_gather()


@functools.partial(
    pl.kernel,
    mesh=_mesh,
    compiler_params=_sc_params,
    out_type=jax.ShapeDtypeStruct((NW, NPAD), jnp.float32),
    scratch_types=[
        pltpu.VMEM((PER_TILE,), jnp.float32),
        pltpu.VMEM((PER_TILE,), jnp.int32),
        pltpu.VMEM((NPAD,), jnp.float32),
    ],
)
def _sc_segmax(logits_hbm, ivi_hbm, out_hbm, lbuf, ibuf, acc):
    wid = _wid()
    base = wid * PER_TILE
    _fill(acc, NPAD, -1e30)
    pltpu.sync_copy(logits_hbm.at[pl.ds(base, PER_TILE)], lbuf)
    pltpu.sync_copy(ivi_hbm.at[pl.ds(base, PER_TILE)], ibuf)

    @pl.loop(0, NCH)
    def _(c):
        iv = ibuf[pl.ds(c * L, L)]
        lv = lbuf[pl.ds(c * L, L)]
        _rmw_scatter(acc, iv, lv, jnp.maximum)

    pltpu.sync_copy(acc, out_hbm.at[wid])


@functools.partial(
    pl.kernel,
    mesh=_mesh,
    compiler_params=_sc_params,
    out_type=[
        jax.ShapeDtypeStruct((EP,), jnp.float32),
        jax.ShapeDtypeStruct((NW, NPAD), jnp.float32),
    ],
    scratch_types=[
        pltpu.VMEM((NPAD,), jnp.float32),
        pltpu.VMEM((PER_TILE,), jnp.float32),
        pltpu.VMEM((PER_TILE,), jnp.int32),
        pltpu.VMEM((PER_TILE,), jnp.float32),
        pltpu.VMEM((NPAD,), jnp.float32),
    ],
)
def _sc_denom(logits_hbm, ivi_hbm, segmax_hbm, ex_hbm, out_hbm,
              smtab, lbuf, ibuf, exbuf, acc):
    wid = _wid()
    base = wid * PER_TILE
    pltpu.sync_copy(segmax_hbm, smtab)
    _fill(acc, NPAD, 0.0)
    pltpu.sync_copy(logits_hbm.at[pl.ds(base, PER_TILE)], lbuf)
    pltpu.sync_copy(ivi_hbm.at[pl.ds(base, PER_TILE)], ibuf)

    @pl.loop(0, NCH)
    def _(c):
        iv = ibuf[pl.ds(c * L, L)]
        lv = lbuf[pl.ds(c * L, L)]
        m = plsc.load_gather(smtab, [iv])
        e = jnp.exp(lv - m)
        exbuf[pl.ds(c * L, L)] = e
        _rmw_scatter(acc, iv, e, jnp.add)

    pltpu.sync_copy(exbuf, ex_hbm.at[pl.ds(base, PER_TILE)])
    pltpu.sync_copy(acc, out_hbm.at[wid])


@functools.partial(
    pl.kernel,
    mesh=_mesh,
    compiler_params=_sc_params,
    out_type=[
        jax.ShapeDtypeStruct((NW, NPAD), jnp.float32),
        jax.ShapeDtypeStruct((NW, NPAD), jnp.float32),
    ],
    scratch_types=[
        pltpu.VMEM((NPAD,), jnp.float32),
        pltpu.VMEM((PER_TILE,), jnp.float32),
        pltpu.VMEM((PER_TILE,), jnp.float32),
        pltpu.VMEM((PER_TILE,), jnp.int32),
        pltpu.VMEM((PER_TILE,), jnp.int32),
        pltpu.VMEM((NPAD,), jnp.float32),
        pltpu.VMEM((NPAD,), jnp.float32),
    ],
)
def _sc_scatter_vj(ex_hbm, na_hbm, ivi_hbm, ivj_hbm, denom_hbm,
                   outs_hbm, outw_hbm,
                   dtab, exb, nab, ibi, ibj, acc_s, acc_w):
    wid = _wid()
    base = wid * PER_TILE
    pltpu.sync_copy(denom_hbm, dtab)
    _fill(acc_s, NPAD, 0.0)
    _fill(acc_w, NPAD, 0.0)
    pltpu.sync_copy(ex_hbm.at[pl.ds(base, PER_TILE)], exb)
    pltpu.sync_copy(na_hbm.at[pl.ds(base, PER_TILE)], nab)
    pltpu.sync_copy(ivi_hbm.at[pl.ds(base, PER_TILE)], ibi)
    pltpu.sync_copy(ivj_hbm.at[pl.ds(base, PER_TILE)], ibj)

    @pl.loop(0, NCH)
    def _(c):
        ivi = ibi[pl.ds(c * L, L)]
        ivj = ibj[pl.ds(c * L, L)]
        ev = exb[pl.ds(c * L, L)]
        nav = nab[pl.ds(c * L, L)]
        d = plsc.load_gather(dtab, [ivi])
        t = ev / (d + 1e-20)
        w = t * nav
        _rmw_scatter(acc_s, ivj, t, jnp.add)
        _rmw_scatter(acc_w, ivj, w, jnp.add)

    pltpu.sync_copy(acc_s, outs_hbm.at[wid])
    pltpu.sync_copy(acc_w, outw_hbm.at[wid])


def kernel(node_attention, memorized_embedding, rel_emb, query_src_emb,
           query_rel_emb, query_time_emb, edge_eg, idx_vi, idx_vj,
           W_proj, b_proj, W_left, b_left, W_right, b_right,
           W_center, b_center, W_lin, b_lin):
    f32 = jnp.float32
    P = W_proj
    Lb = [W_left[:, i * DS:(i + 1) * DS] for i in range(5)]
    Rb = [W_right[:, i * DS:(i + 1) * DS] for i in range(5)]

    w_node = jnp.concatenate([P.T @ Lb[0].T, P.T @ Rb[0].T, W_lin.T], axis=1)
    b_node = jnp.concatenate(
        [jnp.zeros((2 * DS,), f32), b_lin]).reshape(1, 2 * DS + D)
    w_rel = jnp.concatenate([P.T @ Lb[1].T, P.T @ Rb[1].T], axis=1)
    w_q = jnp.concatenate([
        jnp.concatenate([P.T @ Lb[2].T, P.T @ Rb[2].T], axis=1),
        jnp.concatenate([P.T @ Lb[3].T, P.T @ Rb[3].T], axis=1),
        jnp.concatenate([P.T @ Lb[4].T, P.T @ Rb[4].T], axis=1),
    ], axis=0)
    c_l = b_left + b_proj @ (Lb[0].T + Lb[1].T + Lb[2].T + Lb[3].T + Lb[4].T)
    c_r = b_right + b_proj @ (Rb[0].T + Rb[1].T + Rb[2].T + Rb[3].T + Rb[4].T)
    c_q = jnp.concatenate([c_l, c_r]).reshape(1, 2 * DS)
    qcat = jnp.concatenate([query_src_emb, query_rel_emb, query_time_emb],
                           axis=1)

    mem_pad = jnp.pad(memorized_embedding, ((0, NPAD - NSUB), (0, 0)))
    pad_e = EP - E
    ivi_p = jnp.concatenate(
        [idx_vi.astype(jnp.int32), jnp.full((pad_e,), NSUB, jnp.int32)])
    ivj_p = jnp.concatenate(
        [idx_vj.astype(jnp.int32), jnp.full((pad_e,), NSUB, jnp.int32)])
    na_p = jnp.concatenate([node_attention, jnp.zeros((pad_e,), f32)])
    eg3d = edge_eg.astype(jnp.int32).reshape(E // 3200, 1, 3200)

    nl_tab, nr_tab, hl_tab = _node_tables(mem_pad, w_node, b_node)
    qlqr = _query_tables(qcat, w_q, c_q)

    pad_h = EP_H - EH
    ivi32 = idx_vi.astype(jnp.int32)
    ivj32 = idx_vj.astype(jnp.int32)
    gs = []
    for h in (0, 1):
        ivi_h = jnp.concatenate([ivi32[h * EH:(h + 1) * EH],
                                 jnp.full((pad_h,), NSUB, jnp.int32)])
        ivj_h = jnp.concatenate([ivj32[h * EH:(h + 1) * EH],
                                 jnp.full((pad_h,), NSUB, jnp.int32)])
        gs.append(_sc_gather_half(nl_tab, nr_tab, ivi_h, ivj_h))
    halves = [
        _logits_half(h, rel_emb, gs[h][0], gs[h][1], eg3d, w_rel, qlqr,
                     W_center.T, b_center.reshape(1, DS))
        for h in (0, 1)
    ]
    logits_p = jnp.concatenate([halves[0], halves[1],
                                jnp.zeros((pad_e,), f32)])

    segmax_parts = _sc_segmax(logits_p, ivi_p)
    segmax = _combine(segmax_parts, jnp.max)
    ex, denom_parts = _sc_denom(logits_p, ivi_p, segmax)
    denom = _combine(denom_parts, jnp.sum)
    parts_s, parts_w = _sc_scatter_vj(ex, na_p, ivi_p, ivj_p, denom)

    out1_p, out2_p = _final(parts_s, parts_w, hl_tab)
    return out1_p[:NSUB], out2_p[:NSUB]
